# baseline (speedup 1.0000x reference)
import jax
import jax.numpy as jnp
from jax import lax
from jax.experimental import pallas as pl
from jax.experimental.pallas import tpu as pltpu
from jax.experimental.pallas import tpu_sc as plsc

_REG = 0.001
_B = 16384
_D = 32
_ROWS_PER_TILE = 128 // _D
_NW = 32
_BPW = _B // _NW
_CHUNK = 128
_NCHUNK = _BPW // _CHUNK
_GROUPS = _CHUNK // 16


def _sc_body(rows_hbm, cols_hbm, rat_hbm, sen_hbm, w_hbm, p_hbm, q_hbm,
             out_hbm, ridx_refs, cidx_refs, rtile_refs, ctile_refs,
             pbufs, qbufs, rat_v, sen_v, w_v, outv, sem):
    wid = lax.axis_index("s") * 2 + lax.axis_index("c")
    base = wid * _BPW

    for j in range(_NCHUNK):
        pltpu.sync_copy(rows_hbm.at[pl.ds(base + j * _CHUNK, _CHUNK)], ridx_refs[j])
        pltpu.sync_copy(cols_hbm.at[pl.ds(base + j * _CHUNK, _CHUNK)], cidx_refs[j])
        for k in range(_CHUNK // 16):
            sl = pl.ds(k * 16, 16)
            ru = ridx_refs[j][sl]
            cu = cidx_refs[j][sl]
            rtile_refs[j][sl] = lax.shift_left(lax.shift_right_logical(ru, _LOG_U), _LOG_M) | (ru & _MMASK)
            ctile_refs[j][sl] = lax.shift_left(lax.shift_right_logical(cu, _LOG_U), _LOG_M) | (cu & _MMASK)

    def fire(c):
        b = c % 2
        return (
            pltpu.async_copy(p_hbm.at[rtile_refs[c]], pbufs[b], sem),
            pltpu.async_copy(q_hbm.at[ctile_refs[c]], qbufs[b], sem),
        )

    inflight = fire(0)

    pltpu.sync_copy(rat_hbm.at[pl.ds(base, _BPW)], rat_v)
    pltpu.sync_copy(sen_hbm.at[pl.ds(base, _BPW)], sen_v)
    pltpu.sync_copy(w_hbm.at[pl.ds(base, _BPW)], w_v)

    lane = lax.iota(jnp.int32, 16)
    zero = jnp.zeros((16,), jnp.float32)
    masks = [lane == j for j in range(16)]
    perms = [lane ^ sh for sh in (8, 4, 2, 1)]

    lossacc = zero
    regacc = zero
    for c in range(_NCHUNK):
        for cp in inflight:
            cp.wait()
        if c + 1 < _NCHUNK:
            nxt = fire(c + 1)
        b = c % 2
        pb, qb = pbufs[b], qbufs[b]

        def group_body(g, carry, c=c, pb=pb, qb=qb):
            lossacc, regacc = carry
            offu_v = (lax.shift_right_logical(ridx_refs[c][pl.ds(g * 16, 16)], _LOG_M) & 3) * _D
            offi_v = (lax.shift_right_logical(cidx_refs[c][pl.ds(g * 16, 16)], _LOG_M) & 3) * _D
            dots = zero
            for j in range(16):
                r = g * 16 + j
                offu = offu_v[j]
                offi = offi_v[j]
                a0 = pb[r, pl.ds(offu, 16)]
                a1 = pb[r, pl.ds(offu + 16, 16)]
                b0 = qb[r, pl.ds(offi, 16)]
                b1 = qb[r, pl.ds(offi + 16, 16)]
                s = a0 * b0 + a1 * b1
                for p in perms:
                    s = s + s.at[p].get(mode="promise_in_bounds")
                dots = jnp.where(masks[j], s, dots)
                regacc = regacc + (a0 * a0 + a1 * a1 + b0 * b0 + b1 * b1)
            off = c * _CHUNK + g * 16
            rat = rat_v[pl.ds(off, 16)]
            sen = sen_v[pl.ds(off, 16)]
            w2 = w_v[pl.ds(off, 16)] - 0.0001
            pr = 1.0 / (1.0 + jnp.exp(-dots))
            e1 = pr - rat
            e2 = pr - sen
            lossacc = lossacc + e1 * e1 * w2 + e2 * e2 * (1.0 - w2)
            return lossacc, regacc

        lossacc, regacc = lax.fori_loop(0, _GROUPS, group_body, (lossacc, regacc))
        if c + 1 < _NCHUNK:
            inflight = nxt

    outv[...] = lossacc + _REG * regacc
    pltpu.sync_copy(outv, out_hbm.at[wid])


_TRANS_U = 8192
_LOG_U = _TRANS_U.bit_length() - 1
_LOG_M = (_TRANS_U // 4).bit_length() - 1
_MMASK = _TRANS_U // 4 - 1


def _tc_transpose_body(x_ref, o_ref):
    x = x_ref[...]
    d, u = x.shape
    xt = x.T
    m = u // 4
    o_ref[...] = jnp.concatenate([xt[k * m : (k + 1) * m, :] for k in range(4)], axis=1)


def _tc_relayout(table):
    v, d = table.shape
    tv = table.T
    grid = (v + _TRANS_U - 1) // _TRANS_U
    rows_per_blk = _TRANS_U * d // 128
    return pl.pallas_call(
        _tc_transpose_body,
        grid=(grid,),
        in_specs=[pl.BlockSpec((d, _TRANS_U), lambda i: (0, i))],
        out_specs=pl.BlockSpec((rows_per_blk, 128), lambda i: (i, 0)),
        out_shape=jax.ShapeDtypeStruct((grid * rows_per_blk, 128), jnp.float32),
        compiler_params=pltpu.CompilerParams(
            dimension_semantics=("parallel",),
            vmem_limit_bytes=100*1024*1024,
        ),
    )(tv)


@jax.jit
def kernel(rows, cols, ratval, senval, wval, P, Q):
    Pt = _tc_relayout(P)
    Qt = _tc_relayout(Q)
    mesh = plsc.VectorSubcoreMesh(
        core_axis_name="c", subcore_axis_name="s", num_cores=2, num_subcores=16
    )
    partials = pl.kernel(
        _sc_body,
        out_type=jax.ShapeDtypeStruct((_NW, 16), jnp.float32),
        mesh=mesh,
        scratch_types=[
            [pltpu.VMEM((_CHUNK,), jnp.int32) for _ in range(_NCHUNK)],
            [pltpu.VMEM((_CHUNK,), jnp.int32) for _ in range(_NCHUNK)],
            [pltpu.VMEM((_CHUNK,), jnp.int32) for _ in range(_NCHUNK)],
            [pltpu.VMEM((_CHUNK,), jnp.int32) for _ in range(_NCHUNK)],
            [pltpu.VMEM((_CHUNK, _ROWS_PER_TILE * _D), jnp.float32) for _ in range(2)],
            [pltpu.VMEM((_CHUNK, _ROWS_PER_TILE * _D), jnp.float32) for _ in range(2)],
            pltpu.VMEM((_BPW,), jnp.float32),
            pltpu.VMEM((_BPW,), jnp.float32),
            pltpu.VMEM((_BPW,), jnp.float32),
            pltpu.VMEM((16,), jnp.float32),
            pltpu.SemaphoreType.DMA,
        ],
    )(rows, cols, ratval, senval, wval, Pt, Qt)
    return jnp.sum(partials)


# speedup vs baseline: 1.5578x; 1.5578x over previous
import jax
import jax.numpy as jnp
from jax import lax
from jax.experimental import pallas as pl
from jax.experimental.pallas import tpu as pltpu
from jax.experimental.pallas import tpu_sc as plsc

_REG = 0.001
_B = 16384
_D = 32
_ROWS_PER_TILE = 128 // _D
_NW = 32
_BPW = _B // _NW
_CHUNK = 128
_NCHUNK = _BPW // _CHUNK
_GROUPS = _CHUNK // 16


def _sc_body(rows_hbm, cols_hbm, rat_hbm, sen_hbm, w_hbm, p_hbm, q_hbm,
             out_hbm, ridx_refs, cidx_refs, rtile_refs, ctile_refs,
             pbufs, qbufs, rat_v, sen_v, w_v, outv, sem):
    wid = lax.axis_index("s") * 2 + lax.axis_index("c")
    base = wid * _BPW

    for j in range(_NCHUNK):
        pltpu.sync_copy(rows_hbm.at[pl.ds(base + j * _CHUNK, _CHUNK)], ridx_refs[j])
        pltpu.sync_copy(cols_hbm.at[pl.ds(base + j * _CHUNK, _CHUNK)], cidx_refs[j])
        for k in range(_CHUNK // 16):
            sl = pl.ds(k * 16, 16)
            ru = ridx_refs[j][sl]
            cu = cidx_refs[j][sl]
            rtile_refs[j][sl] = lax.shift_left(lax.shift_right_logical(ru, _LOG_U), _LOG_M) | (ru & _MMASK)
            ctile_refs[j][sl] = lax.shift_left(lax.shift_right_logical(cu, _LOG_U), _LOG_M) | (cu & _MMASK)

    def fire(c):
        b = c % 2
        return (
            pltpu.async_copy(p_hbm.at[rtile_refs[c]], pbufs[b], sem),
            pltpu.async_copy(q_hbm.at[ctile_refs[c]], qbufs[b], sem),
        )

    inflight = fire(0)

    pltpu.sync_copy(rat_hbm.at[pl.ds(base, _BPW)], rat_v)
    pltpu.sync_copy(sen_hbm.at[pl.ds(base, _BPW)], sen_v)
    pltpu.sync_copy(w_hbm.at[pl.ds(base, _BPW)], w_v)

    lane = lax.iota(jnp.int32, 16)
    zero = jnp.zeros((16,), jnp.float32)
    masks = [lane == j for j in range(16)]
    perms = [lane ^ sh for sh in (8, 4, 2, 1)]

    lossacc = zero
    regacc = zero
    for c in range(_NCHUNK):
        for cp in inflight:
            cp.wait()
        if c + 1 < _NCHUNK:
            nxt = fire(c + 1)
        b = c % 2
        pb, qb = pbufs[b], qbufs[b]

        def group_body(g, carry, c=c, pb=pb, qb=qb):
            lossacc, regacc = carry
            offu_v = (lax.shift_right_logical(ridx_refs[c][pl.ds(g * 16, 16)], _LOG_M) & 3) * _D
            offi_v = (lax.shift_right_logical(cidx_refs[c][pl.ds(g * 16, 16)], _LOG_M) & 3) * _D
            dots = zero
            for j in range(16):
                r = g * 16 + j
                offu = offu_v[j]
                offi = offi_v[j]
                a0 = pb[r, pl.ds(offu, 16)]
                a1 = pb[r, pl.ds(offu + 16, 16)]
                b0 = qb[r, pl.ds(offi, 16)]
                b1 = qb[r, pl.ds(offi + 16, 16)]
                s = a0 * b0 + a1 * b1
                for p in perms:
                    s = s + s.at[p].get(mode="promise_in_bounds")
                dots = jnp.where(masks[j], s, dots)
                regacc = regacc + (a0 * a0 + a1 * a1 + b0 * b0 + b1 * b1)
            off = c * _CHUNK + g * 16
            rat = rat_v[pl.ds(off, 16)]
            sen = sen_v[pl.ds(off, 16)]
            w2 = w_v[pl.ds(off, 16)] - 0.0001
            pr = 1.0 / (1.0 + jnp.exp(-dots))
            e1 = pr - rat
            e2 = pr - sen
            lossacc = lossacc + e1 * e1 * w2 + e2 * e2 * (1.0 - w2)
            return lossacc, regacc

        lossacc, regacc = lax.fori_loop(0, _GROUPS, group_body, (lossacc, regacc))
        if c + 1 < _NCHUNK:
            inflight = nxt

    outv[...] = lossacc + _REG * regacc
    pltpu.sync_copy(outv, out_hbm.at[wid])


_TRANS_U = 16384
_LOG_U = _TRANS_U.bit_length() - 1
_LOG_M = (_TRANS_U // 4).bit_length() - 1
_MMASK = _TRANS_U // 4 - 1


def _tc_transpose_body(x_ref, o_ref):
    x = x_ref[...]
    d, u = x.shape
    m = u // 4
    acc = jnp.zeros((m, 4 * d), jnp.float32)
    for k in range(4):
        ek = jnp.eye(d, 4 * d, k=k * d, dtype=jnp.float32)
        acc = acc + lax.dot_general(
            x[:, k * m : (k + 1) * m], ek,
            (((0,), (0,)), ((), ())), preferred_element_type=jnp.float32)
    o_ref[...] = acc


def _tc_relayout(table):
    v, d = table.shape
    tv = table.T
    grid = (v + _TRANS_U - 1) // _TRANS_U
    rows_per_blk = _TRANS_U * d // 128
    return pl.pallas_call(
        _tc_transpose_body,
        grid=(grid,),
        in_specs=[pl.BlockSpec((d, _TRANS_U), lambda i: (0, i))],
        out_specs=pl.BlockSpec((rows_per_blk, 128), lambda i: (i, 0)),
        out_shape=jax.ShapeDtypeStruct((grid * rows_per_blk, 128), jnp.float32),
        compiler_params=pltpu.CompilerParams(
            dimension_semantics=("parallel",),
            vmem_limit_bytes=100*1024*1024,
        ),
    )(tv)


@jax.jit
def kernel(rows, cols, ratval, senval, wval, P, Q):
    Pt = _tc_relayout(P)
    Qt = _tc_relayout(Q)
    mesh = plsc.VectorSubcoreMesh(
        core_axis_name="c", subcore_axis_name="s", num_cores=2, num_subcores=16
    )
    partials = pl.kernel(
        _sc_body,
        out_type=jax.ShapeDtypeStruct((_NW, 16), jnp.float32),
        mesh=mesh,
        scratch_types=[
            [pltpu.VMEM((_CHUNK,), jnp.int32) for _ in range(_NCHUNK)],
            [pltpu.VMEM((_CHUNK,), jnp.int32) for _ in range(_NCHUNK)],
            [pltpu.VMEM((_CHUNK,), jnp.int32) for _ in range(_NCHUNK)],
            [pltpu.VMEM((_CHUNK,), jnp.int32) for _ in range(_NCHUNK)],
            [pltpu.VMEM((_CHUNK, _ROWS_PER_TILE * _D), jnp.float32) for _ in range(2)],
            [pltpu.VMEM((_CHUNK, _ROWS_PER_TILE * _D), jnp.float32) for _ in range(2)],
            pltpu.VMEM((_BPW,), jnp.float32),
            pltpu.VMEM((_BPW,), jnp.float32),
            pltpu.VMEM((_BPW,), jnp.float32),
            pltpu.VMEM((16,), jnp.float32),
            pltpu.SemaphoreType.DMA,
        ],
    )(rows, cols, ratval, senval, wval, Pt, Qt)
    return jnp.sum(partials)


# speedup vs baseline: 1.6306x; 1.0467x over previous
import jax
import jax.numpy as jnp
from jax import lax
from jax.experimental import pallas as pl
from jax.experimental.pallas import tpu as pltpu
from jax.experimental.pallas import tpu_sc as plsc

_REG = 0.001
_B = 16384
_D = 32
_ROWS_PER_TILE = 128 // _D
_NW = 32
_BPW = _B // _NW
_CHUNK = 128
_NCHUNK = _BPW // _CHUNK
_GROUPS = _CHUNK // 16


def _sc_body(rows_hbm, cols_hbm, rat_hbm, sen_hbm, w_hbm, p_hbm, q_hbm,
             out_hbm, ridx_refs, cidx_refs, rtile_refs, ctile_refs,
             pbufs, qbufs, rat_v, sen_v, w_v, outv, sem):
    wid = lax.axis_index("s") * 2 + lax.axis_index("c")
    base = wid * _BPW

    for j in range(_NCHUNK):
        pltpu.sync_copy(rows_hbm.at[pl.ds(base + j * _CHUNK, _CHUNK)], ridx_refs[j])
        pltpu.sync_copy(cols_hbm.at[pl.ds(base + j * _CHUNK, _CHUNK)], cidx_refs[j])
        for k in range(_CHUNK // 16):
            sl = pl.ds(k * 16, 16)
            ru = ridx_refs[j][sl]
            cu = cidx_refs[j][sl]
            rtile_refs[j][sl] = lax.shift_left(lax.shift_right_logical(ru, _LOG_U), _LOG_M) | (ru & _MMASK)
            ctile_refs[j][sl] = lax.shift_left(lax.shift_right_logical(cu, _LOG_U), _LOG_M) | (cu & _MMASK)

    def fire(c):
        b = c % 2
        return (
            pltpu.async_copy(p_hbm.at[rtile_refs[c]], pbufs[b], sem),
            pltpu.async_copy(q_hbm.at[ctile_refs[c]], qbufs[b], sem),
        )

    inflight = fire(0)

    pltpu.sync_copy(rat_hbm.at[pl.ds(base, _BPW)], rat_v)
    pltpu.sync_copy(sen_hbm.at[pl.ds(base, _BPW)], sen_v)
    pltpu.sync_copy(w_hbm.at[pl.ds(base, _BPW)], w_v)

    lane = lax.iota(jnp.int32, 16)
    zero = jnp.zeros((16,), jnp.float32)
    masks = [lane == j for j in range(16)]
    perms = [lane ^ sh for sh in (8, 4, 2, 1)]

    lossacc = zero
    regacc = zero
    for c in range(_NCHUNK):
        for cp in inflight:
            cp.wait()
        if c + 1 < _NCHUNK:
            nxt = fire(c + 1)
        b = c % 2
        pb, qb = pbufs[b], qbufs[b]

        def group_body(g, carry, c=c, pb=pb, qb=qb):
            lossacc, regacc = carry
            offu_v = (lax.shift_right_logical(ridx_refs[c][pl.ds(g * 16, 16)], _LOG_M) & 3) * _D
            offi_v = (lax.shift_right_logical(cidx_refs[c][pl.ds(g * 16, 16)], _LOG_M) & 3) * _D
            dots = zero
            for j in range(16):
                r = g * 16 + j
                offu = offu_v[j]
                offi = offi_v[j]
                a0 = pb[r, pl.ds(offu, 16)]
                a1 = pb[r, pl.ds(offu + 16, 16)]
                b0 = qb[r, pl.ds(offi, 16)]
                b1 = qb[r, pl.ds(offi + 16, 16)]
                s = a0 * b0 + a1 * b1
                for p in perms:
                    s = s + s.at[p].get(mode="promise_in_bounds")
                dots = jnp.where(masks[j], s, dots)
                regacc = regacc + (a0 * a0 + a1 * a1 + b0 * b0 + b1 * b1)
            off = c * _CHUNK + g * 16
            rat = rat_v[pl.ds(off, 16)]
            sen = sen_v[pl.ds(off, 16)]
            w2 = w_v[pl.ds(off, 16)] - 0.0001
            pr = 1.0 / (1.0 + jnp.exp(-dots))
            e1 = pr - rat
            e2 = pr - sen
            lossacc = lossacc + e1 * e1 * w2 + e2 * e2 * (1.0 - w2)
            return lossacc, regacc

        lossacc, regacc = lax.fori_loop(0, _GROUPS, group_body, (lossacc, regacc))
        if c + 1 < _NCHUNK:
            inflight = nxt

    outv[...] = lossacc + _REG * regacc
    pltpu.sync_copy(outv, out_hbm.at[wid])


_TRANS_U = 32768
_LOG_U = _TRANS_U.bit_length() - 1
_LOG_M = (_TRANS_U // 4).bit_length() - 1
_MMASK = _TRANS_U // 4 - 1


def _tc_transpose_body(x_ref, o_ref):
    x = x_ref[...]
    d, u = x.shape
    m = u // 4
    acc = jnp.zeros((m, 4 * d), jnp.float32)
    for k in range(4):
        ek = jnp.eye(d, 4 * d, k=k * d, dtype=jnp.float32)
        acc = acc + lax.dot_general(
            x[:, k * m : (k + 1) * m], ek,
            (((0,), (0,)), ((), ())), preferred_element_type=jnp.float32)
    o_ref[...] = acc


def _tc_relayout(table):
    v, d = table.shape
    tv = table.T
    grid = (v + _TRANS_U - 1) // _TRANS_U
    rows_per_blk = _TRANS_U * d // 128
    return pl.pallas_call(
        _tc_transpose_body,
        grid=(grid,),
        in_specs=[pl.BlockSpec((d, _TRANS_U), lambda i: (0, i))],
        out_specs=pl.BlockSpec((rows_per_blk, 128), lambda i: (i, 0)),
        out_shape=jax.ShapeDtypeStruct((grid * rows_per_blk, 128), jnp.float32),
        compiler_params=pltpu.CompilerParams(
            dimension_semantics=("parallel",),
            vmem_limit_bytes=100*1024*1024,
        ),
    )(tv)


@jax.jit
def kernel(rows, cols, ratval, senval, wval, P, Q):
    Pt = _tc_relayout(P)
    Qt = _tc_relayout(Q)
    mesh = plsc.VectorSubcoreMesh(
        core_axis_name="c", subcore_axis_name="s", num_cores=2, num_subcores=16
    )
    partials = pl.kernel(
        _sc_body,
        out_type=jax.ShapeDtypeStruct((_NW, 16), jnp.float32),
        mesh=mesh,
        scratch_types=[
            [pltpu.VMEM((_CHUNK,), jnp.int32) for _ in range(_NCHUNK)],
            [pltpu.VMEM((_CHUNK,), jnp.int32) for _ in range(_NCHUNK)],
            [pltpu.VMEM((_CHUNK,), jnp.int32) for _ in range(_NCHUNK)],
            [pltpu.VMEM((_CHUNK,), jnp.int32) for _ in range(_NCHUNK)],
            [pltpu.VMEM((_CHUNK, _ROWS_PER_TILE * _D), jnp.float32) for _ in range(2)],
            [pltpu.VMEM((_CHUNK, _ROWS_PER_TILE * _D), jnp.float32) for _ in range(2)],
            pltpu.VMEM((_BPW,), jnp.float32),
            pltpu.VMEM((_BPW,), jnp.float32),
            pltpu.VMEM((_BPW,), jnp.float32),
            pltpu.VMEM((16,), jnp.float32),
            pltpu.SemaphoreType.DMA,
        ],
    )(rows, cols, ratval, senval, wval, Pt, Qt)
    return jnp.sum(partials)
